# Initial kernel scaffold; baseline (speedup 1.0000x reference)
import jax
import jax.numpy as jnp
from jax.experimental import pallas as pl


def kernel(h, m_flux, v, edges, W, b):
    raise NotImplementedError("write your pallas kernel here")


# speedup vs baseline: 9.2372x; 9.2372x over previous
import functools

import jax
import jax.numpy as jnp
from jax import lax
from jax.experimental import pallas as pl
from jax.experimental.pallas import tpu as pltpu
from jax.experimental.pallas import tpu_sc as plsc

_N, _E, _D, _DM, _DOUT = 10000, 640000, 128, 16, 16


def _node_proj_body(h_ref, v_ref, wh_ref, wvs_ref, wvd_ref, b_ref, a_ref, bt_ref):
    hW = jnp.dot(h_ref[...], wh_ref[...], preferred_element_type=jnp.float32)
    a_ref[...] = hW + jnp.dot(v_ref[...], wvs_ref[...],
                              preferred_element_type=jnp.float32)
    bt_ref[...] = (hW + jnp.dot(v_ref[...], wvd_ref[...],
                                preferred_element_type=jnp.float32)
                   + b_ref[...])


_NODE_BLK = 2000

_node_proj = pl.pallas_call(
    _node_proj_body,
    grid=(_N // _NODE_BLK,),
    in_specs=[
        pl.BlockSpec((_NODE_BLK, _D), lambda i: (i, 0)),
        pl.BlockSpec((_NODE_BLK, _D), lambda i: (i, 0)),
        pl.BlockSpec((_D, _DOUT), lambda i: (0, 0)),
        pl.BlockSpec((_D, _DOUT), lambda i: (0, 0)),
        pl.BlockSpec((_D, _DOUT), lambda i: (0, 0)),
        pl.BlockSpec((1, _DOUT), lambda i: (0, 0)),
    ],
    out_specs=[
        pl.BlockSpec((_NODE_BLK, _DOUT), lambda i: (i, 0)),
        pl.BlockSpec((_NODE_BLK, _DOUT), lambda i: (i, 0)),
    ],
    out_shape=[
        jax.ShapeDtypeStruct((_N, _DOUT), jnp.float32),
        jax.ShapeDtypeStruct((_N, _DOUT), jnp.float32),
    ],
)


def _flux_proj_body(m_ref, wm_ref, o_ref):
    o_ref[...] = jnp.dot(m_ref[...], wm_ref[...],
                         preferred_element_type=jnp.float32)


_FLUX_BLK = 12800

_flux_proj = pl.pallas_call(
    _flux_proj_body,
    grid=(_E // _FLUX_BLK,),
    in_specs=[
        pl.BlockSpec((_FLUX_BLK, _DM), lambda i: (i, 0)),
        pl.BlockSpec((_DM, _DOUT), lambda i: (0, 0)),
    ],
    out_specs=pl.BlockSpec((_FLUX_BLK, _DOUT), lambda i: (i, 0)),
    out_shape=jax.ShapeDtypeStruct((_E, _DOUT), jnp.float32),
)


_NW = 32
_EPW = _E // _NW
_CHUNK = 2000
_NCHUNK = _EPW // _CHUNK

_sc_mesh = plsc.VectorSubcoreMesh(core_axis_name="c", subcore_axis_name="s")


@functools.partial(
    pl.kernel,
    out_type=jax.ShapeDtypeStruct((_E, _DOUT), jnp.float32),
    mesh=_sc_mesh,
    scratch_types=[
        pltpu.VMEM((_CHUNK,), jnp.int32),
        pltpu.VMEM((_CHUNK,), jnp.int32),
        pltpu.VMEM((_CHUNK, _DOUT), jnp.float32),
        pltpu.VMEM((_CHUNK, _DOUT), jnp.float32),
        pltpu.VMEM((_CHUNK, _DOUT), jnp.float32),
        pltpu.SemaphoreType.DMA,
        pltpu.SemaphoreType.DMA,
    ],
    compiler_params=pltpu.CompilerParams(use_tc_tiling_on_sc=False),
)
def _sc_edge_combine(a_hbm, b_hbm, mw_hbm, src_hbm, dst_hbm, out_hbm,
                     idx_s, idx_d, buf_a, buf_b, buf_m, sem_a, sem_b):
    wid = lax.axis_index("s") * 2 + lax.axis_index("c")

    def chunk_body(ci, carry):
        base = wid * _EPW + ci * _CHUNK
        pltpu.sync_copy(src_hbm.at[pl.ds(base, _CHUNK)], idx_s)
        pltpu.sync_copy(dst_hbm.at[pl.ds(base, _CHUNK)], idx_d)
        cp_a = pltpu.async_copy(a_hbm.at[idx_s], buf_a, sem_a)
        cp_b = pltpu.async_copy(b_hbm.at[idx_d], buf_b, sem_b)
        pltpu.sync_copy(mw_hbm.at[pl.ds(base, _CHUNK)], buf_m)
        cp_a.wait()
        cp_b.wait()

        def row_body(j, c2):
            buf_m[j, :] = buf_m[j, :] + buf_a[j, :] + buf_b[j, :]
            return c2

        lax.fori_loop(0, _CHUNK, row_body, 0)
        pltpu.sync_copy(buf_m, out_hbm.at[pl.ds(base, _CHUNK)])
        return carry

    lax.fori_loop(0, _NCHUNK, chunk_body, 0)


def kernel(h, m_flux, v, edges, W, b):
    wh = W[:_D]
    wm = W[_D:_D + _DM]
    wvs = W[_D + _DM:_D + _DM + _D]
    wvd = W[_D + _DM + _D:]
    a_tab, b_tab = _node_proj(h, v, wh, wvs, wvd, b.reshape(1, _DOUT))
    mw = _flux_proj(m_flux, wm)
    src = edges[:, 0]
    dst = edges[:, 1]
    return _sc_edge_combine(a_tab, b_tab, mw, src, dst)


# speedup vs baseline: 12.3777x; 1.3400x over previous
import functools

import jax
import jax.numpy as jnp
from jax import lax
from jax.experimental import pallas as pl
from jax.experimental.pallas import tpu as pltpu
from jax.experimental.pallas import tpu_sc as plsc

_N, _E, _D, _DM, _DOUT = 10000, 640000, 128, 16, 16


def _node_proj_body(h_ref, v_ref, wh_ref, wvs_ref, wvd_ref, b_ref, a_ref, bt_ref):
    hW = jnp.dot(h_ref[...], wh_ref[...], preferred_element_type=jnp.float32)
    a_ref[...] = hW + jnp.dot(v_ref[...], wvs_ref[...],
                              preferred_element_type=jnp.float32)
    bt_ref[...] = (hW + jnp.dot(v_ref[...], wvd_ref[...],
                                preferred_element_type=jnp.float32)
                   + b_ref[...])


_NODE_BLK = 2000

_node_proj = pl.pallas_call(
    _node_proj_body,
    grid=(_N // _NODE_BLK,),
    in_specs=[
        pl.BlockSpec((_NODE_BLK, _D), lambda i: (i, 0)),
        pl.BlockSpec((_NODE_BLK, _D), lambda i: (i, 0)),
        pl.BlockSpec((_D, _DOUT), lambda i: (0, 0)),
        pl.BlockSpec((_D, _DOUT), lambda i: (0, 0)),
        pl.BlockSpec((_D, _DOUT), lambda i: (0, 0)),
        pl.BlockSpec((1, _DOUT), lambda i: (0, 0)),
    ],
    out_specs=[
        pl.BlockSpec((_NODE_BLK, _DOUT), lambda i: (i, 0)),
        pl.BlockSpec((_NODE_BLK, _DOUT), lambda i: (i, 0)),
    ],
    out_shape=[
        jax.ShapeDtypeStruct((_N, _DOUT), jnp.float32),
        jax.ShapeDtypeStruct((_N, _DOUT), jnp.float32),
    ],
)


_NW = 32
_CHUNK = 1600
_NCHUNKS = _E // _CHUNK
_CPW = -(-_NCHUNKS // _NW)
_P = _CHUNK // 8

_sc_mesh = plsc.VectorSubcoreMesh(core_axis_name="c", subcore_axis_name="s")


@functools.partial(
    pl.kernel,
    out_type=jax.ShapeDtypeStruct((_E // 8, 8 * _DOUT), jnp.float32),
    mesh=_sc_mesh,
    scratch_types=[
        pltpu.VMEM((_CHUNK,), jnp.int32),
        pltpu.VMEM((_CHUNK,), jnp.int32),
        pltpu.VMEM((_CHUNK, _DOUT), jnp.float32),
        pltpu.VMEM((_CHUNK, _DOUT), jnp.float32),
        pltpu.VMEM((_P, 8 * _DOUT), jnp.float32),
        pltpu.SemaphoreType.DMA,
        pltpu.SemaphoreType.DMA,
    ],
    compiler_params=pltpu.CompilerParams(use_tc_tiling_on_sc=False),
)
def _sc_gather_sum(a_hbm, b_hbm, src_hbm, dst_hbm, out_hbm,
                   idx_s, idx_d, buf_a, buf_b, buf_o, sem_a, sem_b):
    wid = lax.axis_index("s") * 2 + lax.axis_index("c")

    def chunk_body(ci, carry):
        chunk = wid + _NW * ci

        @pl.when(chunk < _NCHUNKS)
        def _():
            base = chunk * _CHUNK
            pltpu.sync_copy(src_hbm.at[pl.ds(base, _CHUNK)], idx_s)
            pltpu.sync_copy(dst_hbm.at[pl.ds(base, _CHUNK)], idx_d)
            cp_a = pltpu.async_copy(a_hbm.at[idx_s], buf_a, sem_a)
            cp_b = pltpu.async_copy(b_hbm.at[idx_d], buf_b, sem_b)
            cp_a.wait()
            cp_b.wait()

            def row_body(q, c2):
                for k in range(8):
                    buf_o[q, pl.ds(k * _DOUT, _DOUT)] = (
                        buf_a[k * _P + q, :] + buf_b[k * _P + q, :])
                return c2

            lax.fori_loop(0, _P, row_body, 0)
            pltpu.sync_copy(buf_o, out_hbm.at[pl.ds(chunk * _P, _P)])

        return carry

    lax.fori_loop(0, _CPW, chunk_body, 0)


_CMB_BLK = _CHUNK


def _combine_body(m_ref, g_ref, wm_ref, o_ref):
    mw = jnp.dot(m_ref[...], wm_ref[...], preferred_element_type=jnp.float32)
    for k in range(8):
        o_ref[k * _P:(k + 1) * _P, :] = (
            mw[k * _P:(k + 1) * _P, :] + g_ref[:, k * _DOUT:(k + 1) * _DOUT])


_combine = pl.pallas_call(
    _combine_body,
    grid=(_E // _CMB_BLK,),
    in_specs=[
        pl.BlockSpec((_CMB_BLK, _DM), lambda i: (i, 0)),
        pl.BlockSpec((_CMB_BLK // 8, 8 * _DOUT), lambda i: (i, 0)),
        pl.BlockSpec((_DM, _DOUT), lambda i: (0, 0)),
    ],
    out_specs=pl.BlockSpec((_CMB_BLK, _DOUT), lambda i: (i, 0)),
    out_shape=jax.ShapeDtypeStruct((_E, _DOUT), jnp.float32),
)


def kernel(h, m_flux, v, edges, W, b):
    wh = W[:_D]
    wm = W[_D:_D + _DM]
    wvs = W[_D + _DM:_D + _DM + _D]
    wvd = W[_D + _DM + _D:]
    a_tab, b_tab = _node_proj(h, v, wh, wvs, wvd, b.reshape(1, _DOUT))
    src = edges[:, 0]
    dst = edges[:, 1]
    gsum = _sc_gather_sum(a_tab, b_tab, src, dst)
    return _combine(m_flux, gsum, wm)


# speedup vs baseline: 15.6400x; 1.2636x over previous
import functools

import jax
import jax.numpy as jnp
from jax import lax
from jax.experimental import pallas as pl
from jax.experimental.pallas import tpu as pltpu
from jax.experimental.pallas import tpu_sc as plsc

_N, _E, _D, _DM, _DOUT = 10000, 640000, 128, 16, 16


def _node_proj_body(h_ref, v_ref, wh_ref, wvs_ref, wvd_ref, b_ref, a_ref, bt_ref):
    hW = jnp.dot(h_ref[...], wh_ref[...], preferred_element_type=jnp.float32)
    a_ref[...] = hW + jnp.dot(v_ref[...], wvs_ref[...],
                              preferred_element_type=jnp.float32)
    bt_ref[...] = (hW + jnp.dot(v_ref[...], wvd_ref[...],
                                preferred_element_type=jnp.float32)
                   + b_ref[...])


_NODE_BLK = 2000

_node_proj = pl.pallas_call(
    _node_proj_body,
    grid=(_N // _NODE_BLK,),
    in_specs=[
        pl.BlockSpec((_NODE_BLK, _D), lambda i: (i, 0)),
        pl.BlockSpec((_NODE_BLK, _D), lambda i: (i, 0)),
        pl.BlockSpec((_D, _DOUT), lambda i: (0, 0)),
        pl.BlockSpec((_D, _DOUT), lambda i: (0, 0)),
        pl.BlockSpec((_D, _DOUT), lambda i: (0, 0)),
        pl.BlockSpec((1, _DOUT), lambda i: (0, 0)),
    ],
    out_specs=[
        pl.BlockSpec((_NODE_BLK, _DOUT), lambda i: (i, 0)),
        pl.BlockSpec((_NODE_BLK, _DOUT), lambda i: (i, 0)),
    ],
    out_shape=[
        jax.ShapeDtypeStruct((_N, _DOUT), jnp.float32),
        jax.ShapeDtypeStruct((_N, _DOUT), jnp.float32),
    ],
)


_NW = 32
_CHUNK = 1600
_NCHUNKS = _E // _CHUNK
_CPW = -(-_NCHUNKS // _NW)
_P = _CHUNK // 8

_sc_mesh = plsc.VectorSubcoreMesh(core_axis_name="c", subcore_axis_name="s")


@functools.partial(
    pl.kernel,
    out_type=jax.ShapeDtypeStruct((_E // 8, 8 * _DOUT), jnp.float32),
    mesh=_sc_mesh,
    scratch_types=[
        pltpu.VMEM((_CHUNK,), jnp.int32),
        pltpu.VMEM((_CHUNK,), jnp.int32),
        pltpu.VMEM((_CHUNK, _DOUT), jnp.float32),
        pltpu.VMEM((_CHUNK, _DOUT), jnp.float32),
        pltpu.VMEM((_P, 8 * _DOUT), jnp.float32),
        pltpu.SemaphoreType.DMA,
        pltpu.SemaphoreType.DMA,
    ],
    compiler_params=pltpu.CompilerParams(use_tc_tiling_on_sc=False),
)
def _sc_gather_sum(a_hbm, b_hbm, src_hbm, dst_hbm, out_hbm,
                   idx_s, idx_d, buf_a, buf_b, buf_o, sem_a, sem_b):
    wid = lax.axis_index("s") * 2 + lax.axis_index("c")

    def chunk_body(ci, carry):
        chunk = wid + _NW * ci

        @pl.when(chunk < _NCHUNKS)
        def _():
            base = chunk * _CHUNK
            pltpu.sync_copy(src_hbm.at[pl.ds(base, _CHUNK)], idx_s)
            pltpu.sync_copy(dst_hbm.at[pl.ds(base, _CHUNK)], idx_d)
            cp_a = pltpu.async_copy(a_hbm.at[idx_s], buf_a, sem_a)
            cp_b = pltpu.async_copy(b_hbm.at[idx_d], buf_b, sem_b)
            cp_a.wait()
            cp_b.wait()

            def row_body(q, c2):
                for k in range(8):
                    buf_o[q, pl.ds(k * _DOUT, _DOUT)] = (
                        buf_a[k * _P + q, :] + buf_b[k * _P + q, :])
                return c2

            lax.fori_loop(0, _P, row_body, 0)
            pltpu.sync_copy(buf_o, out_hbm.at[pl.ds(chunk * _P, _P)])

        return carry

    lax.fori_loop(0, _CPW, chunk_body, 0)


_CMB_SUB = 8
_CMB_BLK = _CHUNK * _CMB_SUB


def _combine_body(m_ref, g_ref, wm_ref, o_ref):
    mw = jnp.dot(m_ref[...], wm_ref[...], preferred_element_type=jnp.float32)
    for kb in range(_CMB_SUB):
        for k in range(8):
            r = kb * _CHUNK + k * _P
            o_ref[r:r + _P, :] = (
                mw[r:r + _P, :]
                + g_ref[kb * _P:(kb + 1) * _P, k * _DOUT:(k + 1) * _DOUT])


_combine = pl.pallas_call(
    _combine_body,
    grid=(_E // _CMB_BLK,),
    in_specs=[
        pl.BlockSpec((_CMB_BLK, _DM), lambda i: (i, 0)),
        pl.BlockSpec((_CMB_BLK // 8, 8 * _DOUT), lambda i: (i, 0)),
        pl.BlockSpec((_DM, _DOUT), lambda i: (0, 0)),
    ],
    out_specs=pl.BlockSpec((_CMB_BLK, _DOUT), lambda i: (i, 0)),
    out_shape=jax.ShapeDtypeStruct((_E, _DOUT), jnp.float32),
)


def kernel(h, m_flux, v, edges, W, b):
    wh = W[:_D]
    wm = W[_D:_D + _DM]
    wvs = W[_D + _DM:_D + _DM + _D]
    wvd = W[_D + _DM + _D:]
    a_tab, b_tab = _node_proj(h, v, wh, wvs, wvd, b.reshape(1, _DOUT))
    src = edges[:, 0]
    dst = edges[:, 1]
    gsum = _sc_gather_sum(a_tab, b_tab, src, dst)
    return _combine(m_flux, gsum, wm)
